# Initial kernel scaffold; baseline (speedup 1.0000x reference)
import jax
import jax.numpy as jnp
from jax.experimental import pallas as pl


def kernel(x, edge_index, edge_attr, W):
    raise NotImplementedError("write your pallas kernel here")


# speedup vs baseline: 4.2388x; 4.2388x over previous
import functools

import jax
import jax.numpy as jnp
from jax import lax
from jax.experimental import pallas as pl
from jax.experimental.pallas import tpu as pltpu
from jax.experimental.pallas import tpu_sc as plsc

IN_NODE = 128
IN_EDGE = 16
OUT_DIM = 16

NW = 32
CHUNK = 1000
SUB = 128


def _node_proj_body(x_ref, wn_ref, psrc_ref, pdst_ref):
    p = jax.lax.dot_general(
        x_ref[...], wn_ref[...], (((1,), (0,)), ((), ())),
        preferred_element_type=jnp.float32)
    psrc_ref[...] = p[:, :OUT_DIM]
    pdst_ref[...] = p[:, OUT_DIM:]


def _node_proj(x, wn):
    n = x.shape[0]
    return pl.pallas_call(
        _node_proj_body,
        out_shape=(
            jax.ShapeDtypeStruct((n, OUT_DIM), jnp.float32),
            jax.ShapeDtypeStruct((n, OUT_DIM), jnp.float32),
        ),
    )(x, wn)


def _edge_proj_body(ea_ref, wb_ref, ep_ref):
    ep_ref[...] = jax.lax.dot_general(
        ea_ref[...], wb_ref[...], (((1,), (0,)), ((), ())),
        preferred_element_type=jnp.float32)


def _edge_proj(ea_view, w_big):
    rows = ea_view.shape[0]
    blk = 4000
    grid = rows // blk
    return pl.pallas_call(
        _edge_proj_body,
        grid=(grid,),
        in_specs=[
            pl.BlockSpec((blk, 128), lambda i: (i, 0)),
            pl.BlockSpec((128, 128), lambda i: (0, 0)),
        ],
        out_specs=pl.BlockSpec((blk, 128), lambda i: (i, 0)),
        out_shape=jax.ShapeDtypeStruct((rows, 128), jnp.float32),
    )(ea_view, w_big)


def _sc_gather_add_body(psrc_hbm, pdst_hbm, src_hbm, dst_hbm, ep_hbm,
                        out_hbm, sidx_v, didx_v, rs_v, rd_v, ep_v,
                        gsem, csem, per_w, n_chunks):
    wid = lax.axis_index("s") * 2 + lax.axis_index("c")
    base = wid * per_w

    def chunk_body(c, _):
        off = base + c * CHUNK
        pltpu.sync_copy(src_hbm.at[pl.ds(off, CHUNK)], sidx_v)
        pltpu.sync_copy(dst_hbm.at[pl.ds(off, CHUNK)], didx_v)
        ep_cp = pltpu.async_copy(ep_hbm.at[pl.ds(off, CHUNK)], ep_v, csem)
        cps = []
        for s0 in range(0, CHUNK, SUB):
            n = min(SUB, CHUNK - s0)
            cps.append(pltpu.async_copy(
                psrc_hbm.at[sidx_v.at[pl.ds(s0, n)]],
                rs_v.at[pl.ds(s0, n)], gsem))
            cps.append(pltpu.async_copy(
                pdst_hbm.at[didx_v.at[pl.ds(s0, n)]],
                rd_v.at[pl.ds(s0, n)], gsem))
        for cp in cps:
            cp.wait()
        ep_cp.wait()

        def row_body(i, _):
            rs_v[i] = rs_v[i] + rd_v[i] + ep_v[i]
            return 0
        lax.fori_loop(0, CHUNK, row_body, 0)
        pltpu.sync_copy(rs_v, out_hbm.at[pl.ds(off, CHUNK)])
        return 0

    lax.fori_loop(0, n_chunks, chunk_body, 0)


def _sc_gather_add(psrc, pdst, src, dst, ep):
    e = src.shape[0]
    per_w = e // NW
    n_chunks = per_w // CHUNK
    mesh = plsc.VectorSubcoreMesh(core_axis_name="c", subcore_axis_name="s")
    body = functools.partial(_sc_gather_add_body, per_w=per_w,
                             n_chunks=n_chunks)
    return pl.kernel(
        body,
        out_type=jax.ShapeDtypeStruct((e, OUT_DIM), jnp.float32),
        mesh=mesh,
        compiler_params=pltpu.CompilerParams(use_tc_tiling_on_sc=False),
        scratch_types=[
            pltpu.VMEM((CHUNK,), jnp.int32),
            pltpu.VMEM((CHUNK,), jnp.int32),
            pltpu.VMEM((CHUNK, OUT_DIM), jnp.float32),
            pltpu.VMEM((CHUNK, OUT_DIM), jnp.float32),
            pltpu.VMEM((CHUNK, OUT_DIM), jnp.float32),
            pltpu.SemaphoreType.DMA,
            pltpu.SemaphoreType.DMA,
        ],
    )(psrc, pdst, src, dst, ep)


def kernel(x, edge_index, edge_attr, W):
    x = x.astype(jnp.float32)
    W = W.astype(jnp.float32)
    src = edge_index[0].astype(jnp.int32)
    dst = edge_index[1].astype(jnp.int32)

    wn = jnp.concatenate(
        [W[:, :IN_NODE].T, W[:, IN_NODE + IN_EDGE:].T], axis=1)
    psrc, pdst = _node_proj(x, wn)

    we_t = W[:, IN_NODE:IN_NODE + IN_EDGE].T
    w_big = jnp.kron(jnp.eye(8, dtype=jnp.float32), we_t)
    ea_view = edge_attr.reshape(-1, 128)
    ep = _edge_proj(ea_view, w_big).reshape(-1, OUT_DIM)

    return _sc_gather_add(psrc, pdst, src, dst, ep)


# speedup vs baseline: 7.6744x; 1.8105x over previous
import functools

import jax
import jax.numpy as jnp
from jax import lax
from jax.experimental import pallas as pl
from jax.experimental.pallas import tpu as pltpu
from jax.experimental.pallas import tpu_sc as plsc

IN_NODE = 128
IN_EDGE = 16
OUT_DIM = 16

NW = 32
CHUNK = 1000
SUB = 128


def _node_proj_body(x_ref, wn_ref, psrc_ref, pdst_ref):
    p = jax.lax.dot_general(
        x_ref[...], wn_ref[...], (((1,), (0,)), ((), ())),
        preferred_element_type=jnp.float32)
    psrc_ref[...] = p[:, :OUT_DIM]
    pdst_ref[...] = p[:, OUT_DIM:]


def _node_proj(x, wn):
    n = x.shape[0]
    blk = n // 5
    return pl.pallas_call(
        _node_proj_body,
        grid=(5,),
        in_specs=[
            pl.BlockSpec((blk, IN_NODE), lambda i: (i, 0)),
            pl.BlockSpec((IN_NODE, 2 * OUT_DIM), lambda i: (0, 0)),
        ],
        out_specs=(
            pl.BlockSpec((blk, OUT_DIM), lambda i: (i, 0)),
            pl.BlockSpec((blk, OUT_DIM), lambda i: (i, 0)),
        ),
        out_shape=(
            jax.ShapeDtypeStruct((n, OUT_DIM), jnp.float32),
            jax.ShapeDtypeStruct((n, OUT_DIM), jnp.float32),
        ),
    )(x, wn)


def _sc_gather_add_body(psrc_hbm, pdst_hbm, src_hbm, dst_hbm,
                        out_hbm, sidx_v, didx_v, rs_v, rd_v, gt_v,
                        gsem, per_w, n_chunks):
    wid = lax.axis_index("s") * 2 + lax.axis_index("c")
    base = wid * per_w
    lane = lax.iota(jnp.int32, 16)

    def chunk_body(c, _):
        off = base + c * CHUNK
        pltpu.sync_copy(src_hbm.at[pl.ds(off, CHUNK)], sidx_v)
        pltpu.sync_copy(dst_hbm.at[pl.ds(off, CHUNK)], didx_v)
        cps = []
        for s0 in range(0, CHUNK, SUB):
            n = min(SUB, CHUNK - s0)
            cps.append(pltpu.async_copy(
                psrc_hbm.at[sidx_v.at[pl.ds(s0, n)]],
                rs_v.at[pl.ds(s0, n)], gsem))
            cps.append(pltpu.async_copy(
                pdst_hbm.at[didx_v.at[pl.ds(s0, n)]],
                rd_v.at[pl.ds(s0, n)], gsem))
        for cp in cps:
            cp.wait()

        def row_body(i, _):
            v = rs_v[i] + rd_v[i]
            col = jnp.full((16,), i, dtype=jnp.int32)
            plsc.store_scatter(gt_v, [lane, col], v)
            return 0
        lax.fori_loop(0, CHUNK, row_body, 0)
        pltpu.sync_copy(gt_v, out_hbm.at[:, pl.ds(off, CHUNK)])
        return 0

    lax.fori_loop(0, n_chunks, chunk_body, 0)


def _sc_gather_add(psrc, pdst, src, dst):
    e = src.shape[0]
    per_w = e // NW
    n_chunks = per_w // CHUNK
    mesh = plsc.VectorSubcoreMesh(core_axis_name="c", subcore_axis_name="s")
    body = functools.partial(_sc_gather_add_body, per_w=per_w,
                             n_chunks=n_chunks)
    return pl.kernel(
        body,
        out_type=jax.ShapeDtypeStruct((OUT_DIM, e), jnp.float32),
        mesh=mesh,
        compiler_params=pltpu.CompilerParams(
            use_tc_tiling_on_sc=False, needs_layout_passes=False),
        scratch_types=[
            pltpu.VMEM((CHUNK,), jnp.int32),
            pltpu.VMEM((CHUNK,), jnp.int32),
            pltpu.VMEM((CHUNK, OUT_DIM), jnp.float32),
            pltpu.VMEM((CHUNK, OUT_DIM), jnp.float32),
            pltpu.VMEM((OUT_DIM, CHUNK), jnp.float32),
            pltpu.SemaphoreType.DMA,
        ],
    )(psrc, pdst, src, dst)


def _final_body(we_ref, eat_ref, g_ref, out_ref):
    ep = jax.lax.dot_general(
        we_ref[...], eat_ref[...], (((1,), (0,)), ((), ())),
        preferred_element_type=jnp.float32)
    out_ref[...] = ep + g_ref[...]


def _final_tc(we, eat, g):
    e = eat.shape[1]
    blk = 3200
    grid = e // blk
    return pl.pallas_call(
        _final_body,
        grid=(grid,),
        in_specs=[
            pl.BlockSpec((IN_EDGE, IN_EDGE), lambda i: (0, 0)),
            pl.BlockSpec((IN_EDGE, blk), lambda i: (0, i)),
            pl.BlockSpec((OUT_DIM, blk), lambda i: (0, i)),
        ],
        out_specs=pl.BlockSpec((OUT_DIM, blk), lambda i: (0, i)),
        out_shape=jax.ShapeDtypeStruct((OUT_DIM, e), jnp.float32),
    )(we, eat, g)


def kernel(x, edge_index, edge_attr, W):
    x = x.astype(jnp.float32)
    W = W.astype(jnp.float32)
    src = edge_index[0].astype(jnp.int32)
    dst = edge_index[1].astype(jnp.int32)
    e = edge_attr.shape[0]

    wn = jnp.concatenate(
        [W[:, :IN_NODE].T, W[:, IN_NODE + IN_EDGE:].T], axis=1)
    psrc, pdst = _node_proj(x, wn)

    g = _sc_gather_add(psrc, pdst, src, dst)

    we = W[:, IN_NODE:IN_NODE + IN_EDGE]
    eat = edge_attr.T
    out_t = _final_tc(we, eat, g)
    return out_t.T


# speedup vs baseline: 8.9528x; 1.1666x over previous
import functools

import jax
import jax.numpy as jnp
from jax import lax
from jax.experimental import pallas as pl
from jax.experimental.pallas import tpu as pltpu
from jax.experimental.pallas import tpu_sc as plsc

IN_NODE = 128
IN_EDGE = 16
OUT_DIM = 16

NW = 32
CHUNK = 1000
SUB = 128


def _node_proj_body(x_ref, wn_ref, psrc_ref, pdst_ref):
    p = jax.lax.dot_general(
        x_ref[...], wn_ref[...], (((1,), (0,)), ((), ())),
        preferred_element_type=jnp.float32)
    psrc_ref[...] = p[:, :OUT_DIM]
    pdst_ref[...] = p[:, OUT_DIM:]


def _node_proj(x, wn):
    n = x.shape[0]
    blk = n // 5
    return pl.pallas_call(
        _node_proj_body,
        grid=(5,),
        in_specs=[
            pl.BlockSpec((blk, IN_NODE), lambda i: (i, 0)),
            pl.BlockSpec((IN_NODE, 2 * OUT_DIM), lambda i: (0, 0)),
        ],
        out_specs=(
            pl.BlockSpec((blk, OUT_DIM), lambda i: (i, 0)),
            pl.BlockSpec((blk, OUT_DIM), lambda i: (i, 0)),
        ),
        out_shape=(
            jax.ShapeDtypeStruct((n, OUT_DIM), jnp.float32),
            jax.ShapeDtypeStruct((n, OUT_DIM), jnp.float32),
        ),
    )(x, wn)


def _sc_gather_add_body(psrc_hbm, pdst_hbm, src_hbm, dst_hbm,
                        out_hbm, sidx_v, didx_v, rs_v, rd_v, gt_v,
                        gsem, per_w, n_chunks):
    wid = lax.axis_index("s") * 2 + lax.axis_index("c")
    base = wid * per_w
    lane = lax.iota(jnp.int32, 16)

    def chunk_body(c, _):
        off = base + c * CHUNK
        pltpu.sync_copy(src_hbm.at[pl.ds(off, CHUNK)], sidx_v)
        pltpu.sync_copy(dst_hbm.at[pl.ds(off, CHUNK)], didx_v)
        cps = []
        for s0 in range(0, CHUNK, SUB):
            n = min(SUB, CHUNK - s0)
            cps.append(pltpu.async_copy(
                psrc_hbm.at[sidx_v.at[pl.ds(s0, n)]],
                rs_v.at[pl.ds(s0, n)], gsem))
            cps.append(pltpu.async_copy(
                pdst_hbm.at[didx_v.at[pl.ds(s0, n)]],
                rd_v.at[pl.ds(s0, n)], gsem))
        for cp in cps:
            cp.wait()

        @plsc.parallel_loop(0, CHUNK, step=8, unroll=4)
        def row_body(i):
            for k in range(8):
                v = rs_v[i + k] + rd_v[i + k]
                col = jnp.full((16,), i + k, dtype=jnp.int32)
                plsc.store_scatter(gt_v, [lane, col], v)
        pltpu.sync_copy(gt_v, out_hbm.at[:, pl.ds(off, CHUNK)])
        return 0

    lax.fori_loop(0, n_chunks, chunk_body, 0)


def _sc_gather_add(psrc, pdst, src, dst):
    e = src.shape[0]
    per_w = e // NW
    n_chunks = per_w // CHUNK
    mesh = plsc.VectorSubcoreMesh(core_axis_name="c", subcore_axis_name="s")
    body = functools.partial(_sc_gather_add_body, per_w=per_w,
                             n_chunks=n_chunks)
    return pl.kernel(
        body,
        out_type=jax.ShapeDtypeStruct((OUT_DIM, e), jnp.float32),
        mesh=mesh,
        compiler_params=pltpu.CompilerParams(
            use_tc_tiling_on_sc=False, needs_layout_passes=False),
        scratch_types=[
            pltpu.VMEM((CHUNK,), jnp.int32),
            pltpu.VMEM((CHUNK,), jnp.int32),
            pltpu.VMEM((CHUNK, OUT_DIM), jnp.float32),
            pltpu.VMEM((CHUNK, OUT_DIM), jnp.float32),
            pltpu.VMEM((OUT_DIM, CHUNK), jnp.float32),
            pltpu.SemaphoreType.DMA,
        ],
    )(psrc, pdst, src, dst)


def _final_body(we_ref, eat_ref, g_ref, out_ref):
    ep = jax.lax.dot_general(
        we_ref[...], eat_ref[...], (((1,), (0,)), ((), ())),
        preferred_element_type=jnp.float32)
    out_ref[...] = ep + g_ref[...]


def _final_tc(we, eat, g):
    e = eat.shape[1]
    blk = 3200
    grid = e // blk
    return pl.pallas_call(
        _final_body,
        grid=(grid,),
        in_specs=[
            pl.BlockSpec((IN_EDGE, IN_EDGE), lambda i: (0, 0)),
            pl.BlockSpec((IN_EDGE, blk), lambda i: (0, i)),
            pl.BlockSpec((OUT_DIM, blk), lambda i: (0, i)),
        ],
        out_specs=pl.BlockSpec((OUT_DIM, blk), lambda i: (0, i)),
        out_shape=jax.ShapeDtypeStruct((OUT_DIM, e), jnp.float32),
    )(we, eat, g)


def kernel(x, edge_index, edge_attr, W):
    x = x.astype(jnp.float32)
    W = W.astype(jnp.float32)
    src = edge_index[0].astype(jnp.int32)
    dst = edge_index[1].astype(jnp.int32)
    e = edge_attr.shape[0]

    wn = jnp.concatenate(
        [W[:, :IN_NODE].T, W[:, IN_NODE + IN_EDGE:].T], axis=1)
    psrc, pdst = _node_proj(x, wn)

    g = _sc_gather_add(psrc, pdst, src, dst)

    we = W[:, IN_NODE:IN_NODE + IN_EDGE]
    eat = edge_attr.T
    out_t = _final_tc(we, eat, g)
    return out_t.T


# speedup vs baseline: 10.9804x; 1.2265x over previous
import functools

import jax
import jax.numpy as jnp
from jax import lax
from jax.experimental import pallas as pl
from jax.experimental.pallas import tpu as pltpu
from jax.experimental.pallas import tpu_sc as plsc

IN_NODE = 128
IN_EDGE = 16
OUT_DIM = 16

NW = 32
CHUNK = 1000
SUB = 128


def _node_proj_body(x_ref, wn_ref, psrc_ref, pdst_ref):
    p = jax.lax.dot_general(
        x_ref[...], wn_ref[...], (((1,), (0,)), ((), ())),
        preferred_element_type=jnp.float32)
    psrc_ref[...] = p[:, :OUT_DIM]
    pdst_ref[...] = p[:, OUT_DIM:]


def _node_proj(x, wn):
    n = x.shape[0]
    blk = n // 5
    return pl.pallas_call(
        _node_proj_body,
        grid=(5,),
        in_specs=[
            pl.BlockSpec((blk, IN_NODE), lambda i: (i, 0)),
            pl.BlockSpec((IN_NODE, 2 * OUT_DIM), lambda i: (0, 0)),
        ],
        out_specs=(
            pl.BlockSpec((blk, OUT_DIM), lambda i: (i, 0)),
            pl.BlockSpec((blk, OUT_DIM), lambda i: (i, 0)),
        ),
        out_shape=(
            jax.ShapeDtypeStruct((n, OUT_DIM), jnp.float32),
            jax.ShapeDtypeStruct((n, OUT_DIM), jnp.float32),
        ),
    )(x, wn)


def _sc_gather_add_body(psrc_hbm, pdst_hbm, src_hbm, dst_hbm,
                        out_hbm, sidx_v, didx_v, rs_v, rd_v, gt_v,
                        gsem, per_w, n_chunks):
    wid = lax.axis_index("s") * 2 + lax.axis_index("c")
    base = wid * per_w
    lane = lax.iota(jnp.int32, 16)

    def chunk_body(c, _):
        off = base + c * CHUNK
        pltpu.sync_copy(src_hbm.at[pl.ds(off, CHUNK)], sidx_v)
        pltpu.sync_copy(dst_hbm.at[pl.ds(off, CHUNK)], didx_v)
        cps = []
        for s0 in range(0, CHUNK, SUB):
            n = min(SUB, CHUNK - s0)
            cps.append(pltpu.async_copy(
                psrc_hbm.at[sidx_v.at[pl.ds(s0, n)]],
                rs_v.at[pl.ds(s0, n)], gsem))
            cps.append(pltpu.async_copy(
                pdst_hbm.at[didx_v.at[pl.ds(s0, n)]],
                rd_v.at[pl.ds(s0, n)], gsem))
        for cp in cps:
            cp.wait()

        @plsc.parallel_loop(0, CHUNK, step=8, unroll=4)
        def row_body(i):
            for k in range(8):
                v = rs_v[i + k] + rd_v[i + k]
                col = jnp.full((16,), i + k, dtype=jnp.int32)
                plsc.store_scatter(gt_v, [lane, col], v)
        pltpu.sync_copy(gt_v, out_hbm.at[:, pl.ds(off, CHUNK)])
        return 0

    lax.fori_loop(0, n_chunks, chunk_body, 0)


def _sc_gather_add(psrc, pdst, src, dst):
    e = src.shape[0]
    per_w = e // NW
    n_chunks = per_w // CHUNK
    mesh = plsc.VectorSubcoreMesh(core_axis_name="c", subcore_axis_name="s")
    body = functools.partial(_sc_gather_add_body, per_w=per_w,
                             n_chunks=n_chunks)
    return pl.kernel(
        body,
        out_type=jax.ShapeDtypeStruct((OUT_DIM, e), jnp.float32),
        mesh=mesh,
        compiler_params=pltpu.CompilerParams(
            use_tc_tiling_on_sc=False, needs_layout_passes=False),
        scratch_types=[
            pltpu.VMEM((CHUNK,), jnp.int32),
            pltpu.VMEM((CHUNK,), jnp.int32),
            pltpu.VMEM((CHUNK, OUT_DIM), jnp.float32),
            pltpu.VMEM((CHUNK, OUT_DIM), jnp.float32),
            pltpu.VMEM((OUT_DIM, CHUNK), jnp.float32),
            pltpu.SemaphoreType.DMA,
        ],
    )(psrc, pdst, src, dst)


def _final_body(we_ref, eat_ref, g_ref, out_ref):
    ep = jax.lax.dot_general(
        we_ref[...], eat_ref[...], (((1,), (0,)), ((), ())),
        preferred_element_type=jnp.float32)
    out_ref[...] = ep + g_ref[...]


def _final_tc(we, eat, g):
    e = eat.shape[1]
    blk = 12800
    grid = e // blk
    return pl.pallas_call(
        _final_body,
        grid=(grid,),
        in_specs=[
            pl.BlockSpec((IN_EDGE, IN_EDGE), lambda i: (0, 0)),
            pl.BlockSpec((IN_EDGE, blk), lambda i: (0, i)),
            pl.BlockSpec((OUT_DIM, blk), lambda i: (0, i)),
        ],
        out_specs=pl.BlockSpec((OUT_DIM, blk), lambda i: (0, i)),
        out_shape=jax.ShapeDtypeStruct((OUT_DIM, e), jnp.float32),
    )(we, eat, g)


def kernel(x, edge_index, edge_attr, W):
    x = x.astype(jnp.float32)
    W = W.astype(jnp.float32)
    src = edge_index[0].astype(jnp.int32)
    dst = edge_index[1].astype(jnp.int32)
    e = edge_attr.shape[0]

    wn = jnp.concatenate(
        [W[:, :IN_NODE].T, W[:, IN_NODE + IN_EDGE:].T], axis=1)
    psrc, pdst = _node_proj(x, wn)

    g = _sc_gather_add(psrc, pdst, src, dst)

    we = W[:, IN_NODE:IN_NODE + IN_EDGE]
    eat = edge_attr.T
    out_t = _final_tc(we, eat, g)
    return out_t.T
